# baseline (speedup 1.0000x reference)
import jax
import jax.numpy as jnp
from jax.experimental import pallas as pl

_NUM_CLASSES = 64
_ROWS = 4000


def _onehot_body(idx_ref, out_ref):
    idx = idx_ref[...]
    iota = jax.lax.broadcasted_iota(jnp.int32, (_ROWS, _NUM_CLASSES), 1)
    out_ref[...] = (iota == idx).astype(jnp.float32)


def kernel(species_index, pos):
    n = species_index.shape[0]
    idx2 = species_index.reshape(n, 1).astype(jnp.int32)
    out = pl.pallas_call(
        _onehot_body,
        grid=(n // _ROWS,),
        in_specs=[pl.BlockSpec((_ROWS, 1), lambda i: (i, 0))],
        out_specs=pl.BlockSpec((_ROWS, _NUM_CLASSES), lambda i: (i, 0)),
        out_shape=jax.ShapeDtypeStruct((n, _NUM_CLASSES), jnp.float32),
    )(idx2)
    return out, out


# speedup vs baseline: 1.6470x; 1.6470x over previous
import jax, jax.numpy as jnp
from jax import lax
from jax.experimental import pallas as pl

_NC = 64
_B = 4096


_S = _B // 128


def _body(idx_ref, out_ref):
    x = idx_ref[0]
    iota = lax.broadcasted_iota(jnp.int32, (128, _NC), 1).astype(jnp.float32)
    sub = lax.broadcasted_iota(jnp.int32, (_S, 128), 0)
    for k in range(_S):
        ek = (sub == k).astype(jnp.float32)
        m = lax.dot_general(x, ek, (((0,), (0,)), ((), ())),
                            preferred_element_type=jnp.float32)
        out_ref[pl.ds(k * 128, 128), :] = (m[:, :_NC] == iota).astype(jnp.float32)


def kernel(species_index, pos):
    n = species_index.shape[0]
    g = (n + _B - 1) // _B
    idx_p = jnp.pad(species_index.astype(jnp.float32), (0, g * _B - n))
    idx3 = idx_p.reshape(g, _S, 128)
    out = pl.pallas_call(
        _body,
        grid=(g,),
        in_specs=[pl.BlockSpec((1, _S, 128), lambda i: (i, 0, 0))],
        out_specs=pl.BlockSpec((_B, _NC), lambda i: (i, 0)),
        out_shape=jax.ShapeDtypeStruct((n, _NC), jnp.float32),
    )(idx3)
    return out, out


# speedup vs baseline: 3.7236x; 2.2608x over previous
import jax, jax.numpy as jnp
from jax import lax
from jax.experimental import pallas as pl

_NC = 64
_C = 2048


def _body(idx_ref, a_ref, b_ref):
    idx = idx_ref[0]
    iota = lax.broadcasted_iota(jnp.int32, (_NC, _C), 0)
    oh = (iota == idx).astype(jnp.float32)
    a_ref[...] = oh
    b_ref[...] = oh


def kernel(species_index, pos):
    n = species_index.shape[0]
    g = (n + _C - 1) // _C
    idx_p = jnp.pad(species_index.astype(jnp.int32), (0, g * _C - n))
    idx3 = idx_p.reshape(g, 1, _C)
    spec = pl.BlockSpec((_NC, _C), lambda i: (0, i))
    a, b = pl.pallas_call(
        _body,
        grid=(g,),
        in_specs=[pl.BlockSpec((1, 1, _C), lambda i: (i, 0, 0))],
        out_specs=[spec, spec],
        out_shape=[jax.ShapeDtypeStruct((_NC, n), jnp.float32)] * 2,
    )(idx3)
    return a.T, b.T


# speedup vs baseline: 6.6752x; 1.7927x over previous
import jax, jax.numpy as jnp
from jax import lax
from jax.experimental import pallas as pl

_NC = 64
_C = 8192


def _body(idx_ref, a_ref, b_ref):
    idx = idx_ref[0]
    iota = lax.broadcasted_iota(jnp.int32, (_NC, _C), 0)
    oh = (iota == idx).astype(jnp.float32)
    a_ref[...] = oh
    b_ref[...] = oh


def kernel(species_index, pos):
    n = species_index.shape[0]
    g = (n + _C - 1) // _C
    idx_p = jnp.pad(species_index.astype(jnp.int32), (0, g * _C - n))
    idx3 = idx_p.reshape(g, 1, _C)
    spec = pl.BlockSpec((_NC, _C), lambda i: (0, i))
    a, b = pl.pallas_call(
        _body,
        grid=(g,),
        in_specs=[pl.BlockSpec((1, 1, _C), lambda i: (i, 0, 0))],
        out_specs=[spec, spec],
        out_shape=[jax.ShapeDtypeStruct((_NC, n), jnp.float32)] * 2,
    )(idx3)
    return a.T, b.T
